# Initial kernel scaffold; baseline (speedup 1.0000x reference)
import jax
import jax.numpy as jnp
from jax.experimental import pallas as pl


def kernel(k, D):
    raise NotImplementedError("write your pallas kernel here")


# speedup vs baseline: 68.7716x; 68.7716x over previous
import functools

import jax
import jax.numpy as jnp
from jax import lax
from jax.experimental import pallas as pl
from jax.experimental.pallas import tpu as pltpu

L = 24
M = 64
N = 1024
S = 8
B = 2048
BB = 512


def _omp_block(kt, Dm, Dt):
    Bb = kt.shape[0]
    f32 = jnp.float32
    bf16 = jnp.bfloat16
    iota = lax.broadcasted_iota(jnp.int32, (Bb, N), 1)
    D16 = Dm.astype(bf16)

    r = kt
    idxs = []
    dcols = []
    rhs = []
    g = [[None] * S for _ in range(S)]
    c = [[None] * S for _ in range(S)]
    alpha0 = None
    x = None
    recon = None
    for t in range(S):
        proj = jnp.dot(r.astype(bf16), D16,
                       preferred_element_type=f32)
        if t == 0:
            alpha0 = proj
        a = jnp.abs(proj)
        m = jnp.max(a, axis=1, keepdims=True)
        idx = jnp.min(jnp.where(a == m, iota, N), axis=1)
        idxs.append(idx)
        onehot = (iota == idx[:, None]).astype(f32)
        dcol = jnp.dot(onehot, Dt, preferred_element_type=f32,
                       precision=lax.Precision.HIGHEST)
        dcols.append(dcol)
        rhs.append(jnp.sum(alpha0 * onehot, axis=1))
        grow = jnp.dot(dcol.astype(bf16), D16, preferred_element_type=f32)
        for j in range(t + 1):
            gtj = jnp.sum(grow * (iota == idxs[j][:, None]).astype(f32),
                          axis=1)
            g[t][j] = gtj
        for j in range(t + 1):
            acc = g[t][j]
            if j == t:
                acc = acc + 1e-6
            for p in range(j):
                acc = acc - c[t][p] * c[j][p]
            if j < t:
                c[t][j] = acc / c[j][j]
            else:
                c[t][t] = jnp.sqrt(acc)
        z = [None] * (t + 1)
        for i in range(t + 1):
            acc = rhs[i]
            for p in range(i):
                acc = acc - c[i][p] * z[p]
            z[i] = acc / c[i][i]
        x = [None] * (t + 1)
        for i in range(t, -1, -1):
            acc = z[i]
            for p in range(i + 1, t + 1):
                acc = acc - c[p][i] * x[p]
            x[i] = acc / c[i][i]
        recon = dcols[0] * x[0][:, None]
        for j in range(1, t + 1):
            recon = recon + dcols[j] * x[j][:, None]
        r = kt - recon
    return idxs, x, recon, r, iota


def _kernel_body(kt_ref, d_ref, dt_ref, y_ref, khat_ref, loss_ref):
    kt = kt_ref[0]
    Dm = d_ref[0]
    Dt = dt_ref[0]
    idxs, x, recon, resid, iota = _omp_block(kt, Dm, Dt)

    y = jnp.zeros((kt.shape[0], N), jnp.float32)
    for t in range(S):
        y = jnp.where(iota == idxs[t][:, None], x[t][:, None], y)
    y_ref[...] = y
    khat_ref[0] = recon

    li = pl.program_id(0)
    bi = pl.program_id(1)

    @pl.when((li == 0) & (bi == 0))
    def _():
        loss_ref[0, 0] = 0.0

    loss_ref[0, 0] += jnp.sum(resid * resid)

    @pl.when((li == L - 1) & (bi == pl.num_programs(1) - 1))
    def _():
        loss_ref[0, 0] = loss_ref[0, 0] / (B * L * M)


@functools.partial(jax.jit, static_argnames=("interpret",))
def kernel(k, D, interpret=False):
    ktr = jnp.transpose(k, (1, 0, 2))
    Dt = jnp.transpose(D, (0, 2, 1))
    nb = B // BB
    y, khat, loss = pl.pallas_call(
        _kernel_body,
        grid=(L, nb),
        in_specs=[
            pl.BlockSpec((1, BB, M), lambda l, b: (l, b, 0)),
            pl.BlockSpec((1, M, N), lambda l, b: (l, 0, 0)),
            pl.BlockSpec((1, N, M), lambda l, b: (l, 0, 0)),
        ],
        out_specs=[
            pl.BlockSpec((BB, N), lambda l, b: (b, l)),
            pl.BlockSpec((1, BB, M), lambda l, b: (l, b, 0)),
            pl.BlockSpec(memory_space=pltpu.SMEM, block_shape=(1, 1),
                         index_map=lambda l, b: (0, 0)),
        ],
        out_shape=[
            jax.ShapeDtypeStruct((B, L * N), jnp.float32),
            jax.ShapeDtypeStruct((L, B, M), jnp.float32),
            jax.ShapeDtypeStruct((1, 1), jnp.float32),
        ],
        interpret=interpret,
    )(ktr, D, Dt)
    return (loss[0, 0], jnp.transpose(khat, (1, 0, 2)),
            jnp.reshape(y, (B, L, N)))
